# Initial kernel scaffold; baseline (speedup 1.0000x reference)
import jax
import jax.numpy as jnp
from jax.experimental import pallas as pl


def kernel(x, gemm_vs, W, b):
    raise NotImplementedError("write your pallas kernel here")


# speedup vs baseline: 1.0140x; 1.0140x over previous
import functools

import jax
import jax.numpy as jnp
from jax import lax
from jax.experimental import pallas as pl
from jax.experimental.pallas import tpu as pltpu
from jax.experimental.pallas import tpu_sc as plsc

NCORES = 2
NSUB = 16
NW = NCORES * NSUB
CB = 128
CBN = 512


def _mm_body(x_ref, w_ref, b_ref, y_ref):
    j = pl.program_id(1)
    acc = lax.dot_general(
        x_ref[0], w_ref[0], (((0,), (0,)), ((), ())),
        preferred_element_type=jnp.float32)
    bias = jnp.where(j == 0, b_ref[...], jnp.zeros_like(b_ref[...]))
    y_ref[...] = acc + bias


def _tr_body(s_ref, o_ref):
    o_ref[0] = s_ref[...].T


def _make_sc_gather(NP, K, C_out, chunks):
    mesh = plsc.VectorSubcoreMesh(core_axis_name="c", subcore_axis_name="s")

    @functools.partial(
        pl.kernel,
        out_type=jax.ShapeDtypeStruct((NP, C_out), jnp.float32),
        mesh=mesh,
        scratch_types=[
            pltpu.VMEM((K - 1, CB), jnp.int32),
            pltpu.VMEM((CB, C_out), jnp.float32),
            pltpu.SemaphoreType.DMA,
        ],
    )
    def sc_gather(y_hbm, idx_hbm, s_hbm, idx_v, acc_v, sem):
        wid = lax.axis_index("s") * NCORES + lax.axis_index("c")

        def chunk(t, carry):
            base = (wid * chunks + t) * CB
            pltpu.sync_copy(idx_hbm.at[:, pl.ds(base, CB)], idx_v)

            def off(kk, c2):
                sl = pl.ds(kk * 16, 16)
                for j in range(K - 1):
                    idx_v[j, sl] = idx_v[j, sl] + jnp.int32((j + 1) * NP)
                return c2

            lax.fori_loop(0, CB // 16, off, 0)
            pltpu.sync_copy(y_hbm.at[pl.ds(base, CB)], acc_v)
            for j in range(K - 1):
                pltpu.async_copy(
                    y_hbm.at[idx_v.at[j]], acc_v, sem, add=True).wait()
            pltpu.sync_copy(acc_v, s_hbm.at[pl.ds(base, CB)])
            return carry

        lax.fori_loop(0, chunks, chunk, 0)

    return sc_gather


def kernel(x, gemm_vs, W, b):
    Bsz, C, N = x.shape
    K = gemm_vs.shape[-1] + 1
    C_out = W.shape[0]
    NP = ((N + NW * CB - 1) // (NW * CB)) * (NW * CB)
    chunks = NP // (NW * CB)
    nblk = pl.cdiv(N, CBN)

    Wr = jnp.transpose(W, (2, 1, 0))
    b2 = b.reshape(1, C_out)
    idxT = jnp.transpose(gemm_vs[0])
    idxT = jnp.pad(idxT, ((0, 0), (0, NP - N)))

    Y = pl.pallas_call(
        _mm_body,
        grid=(nblk, K),
        in_specs=[
            pl.BlockSpec((1, C, CBN), lambda nb, j: (0, 0, nb)),
            pl.BlockSpec((1, C, C_out), lambda nb, j: (j, 0, 0)),
            pl.BlockSpec((1, C_out), lambda nb, j: (0, 0)),
        ],
        out_specs=pl.BlockSpec((CBN, C_out), lambda nb, j: (j * (NP // CBN) + nb, 0)),
        out_shape=jax.ShapeDtypeStruct((K * NP, C_out), jnp.float32),
    )(x, Wr, b2)

    S = _make_sc_gather(NP, K, C_out, chunks)(Y, idxT)

    out = pl.pallas_call(
        _tr_body,
        grid=(nblk,),
        in_specs=[pl.BlockSpec((CBN, C_out), lambda nb: (nb, 0))],
        out_specs=pl.BlockSpec((1, C_out, CBN), lambda nb: (0, 0, nb)),
        out_shape=jax.ShapeDtypeStruct((1, C_out, N), jnp.float32),
    )(S)

    return out[..., None]


# speedup vs baseline: 1.2750x; 1.2574x over previous
import functools

import jax
import jax.numpy as jnp
from jax import lax
from jax.experimental import pallas as pl
from jax.experimental.pallas import tpu as pltpu
from jax.experimental.pallas import tpu_sc as plsc

NCORES = 2
NSUB = 16
NW = NCORES * NSUB
CB = 128
CBN = 512


def _mm_body(x_ref, w_ref, b_ref, y_ref):
    j = pl.program_id(1)
    acc = lax.dot_general(
        x_ref[0], w_ref[0], (((0,), (0,)), ((), ())),
        preferred_element_type=jnp.float32)
    bias = jnp.where(j == 0, b_ref[...], jnp.zeros_like(b_ref[...]))
    y_ref[...] = acc + bias


def _tr_body(s_ref, o_ref):
    o_ref[0] = s_ref[...].T


def _make_sc_gather(NP, K, C_out, chunks):
    mesh = plsc.VectorSubcoreMesh(core_axis_name="c", subcore_axis_name="s")

    @functools.partial(
        pl.kernel,
        out_type=jax.ShapeDtypeStruct((NP, C_out), jnp.float32),
        mesh=mesh,
        scratch_types=[
            pltpu.VMEM((K - 1, CB), jnp.int32),
            pltpu.VMEM((CB, C_out), jnp.float32),
            pltpu.SemaphoreType.DMA,
        ],
    )
    def sc_gather(y_hbm, idx_hbm, s_hbm, idx_v, acc_v, sem):
        wid = lax.axis_index("s") * NCORES + lax.axis_index("c")

        def chunk(t, carry):
            base = (wid * chunks + t) * CB
            pltpu.sync_copy(idx_hbm.at[:, pl.ds(base, CB)], idx_v)

            def off(kk, c2):
                sl = pl.ds(kk * 16, 16)
                for j in range(K - 1):
                    idx_v[j, sl] = idx_v[j, sl] + jnp.int32((j + 1) * NP)
                return c2

            lax.fori_loop(0, CB // 16, off, 0)
            pltpu.sync_copy(y_hbm.at[pl.ds(base, CB)], acc_v)
            copies = [
                pltpu.async_copy(y_hbm.at[idx_v.at[j]], acc_v, sem, add=True)
                for j in range(K - 1)
            ]
            for c in copies:
                c.wait()
            pltpu.sync_copy(acc_v, s_hbm.at[pl.ds(base, CB)])
            return carry

        lax.fori_loop(0, chunks, chunk, 0)

    return sc_gather


def kernel(x, gemm_vs, W, b):
    Bsz, C, N = x.shape
    K = gemm_vs.shape[-1] + 1
    C_out = W.shape[0]
    NP = ((N + NW * CB - 1) // (NW * CB)) * (NW * CB)
    chunks = NP // (NW * CB)
    nblk = pl.cdiv(N, CBN)

    Wr = jnp.transpose(W, (2, 1, 0))
    b2 = b.reshape(1, C_out)
    idxT = jnp.transpose(gemm_vs[0])
    idxT = jnp.pad(idxT, ((0, 0), (0, NP - N)))

    Y = pl.pallas_call(
        _mm_body,
        grid=(nblk, K),
        in_specs=[
            pl.BlockSpec((1, C, CBN), lambda nb, j: (0, 0, nb)),
            pl.BlockSpec((1, C, C_out), lambda nb, j: (j, 0, 0)),
            pl.BlockSpec((1, C_out), lambda nb, j: (0, 0)),
        ],
        out_specs=pl.BlockSpec((CBN, C_out), lambda nb, j: (j * (NP // CBN) + nb, 0)),
        out_shape=jax.ShapeDtypeStruct((K * NP, C_out), jnp.float32),
    )(x, Wr, b2)

    S = _make_sc_gather(NP, K, C_out, chunks)(Y, idxT)

    out = pl.pallas_call(
        _tr_body,
        grid=(nblk,),
        in_specs=[pl.BlockSpec((CBN, C_out), lambda nb: (nb, 0))],
        out_specs=pl.BlockSpec((1, C_out, CBN), lambda nb: (0, 0, nb)),
        out_shape=jax.ShapeDtypeStruct((1, C_out, N), jnp.float32),
    )(S)

    return out[..., None]


# speedup vs baseline: 1.3834x; 1.0850x over previous
import functools

import jax
import jax.numpy as jnp
from jax import lax
from jax.experimental import pallas as pl
from jax.experimental.pallas import tpu as pltpu
from jax.experimental.pallas import tpu_sc as plsc

NCORES = 2
NSUB = 16
NW = NCORES * NSUB
CB = 128
CBN = 512


def _mm_body(x_ref, w_ref, b_ref, y_ref):
    j = pl.program_id(1)
    acc = lax.dot_general(
        x_ref[0].astype(jnp.bfloat16), w_ref[0], (((0,), (0,)), ((), ())),
        preferred_element_type=jnp.float32)
    bias = jnp.where(j == 0, b_ref[...], jnp.zeros_like(b_ref[...]))
    y_ref[...] = acc + bias


def _tr_body(s_ref, o_ref):
    o_ref[0] = s_ref[...].T


def _make_sc_gather(NP, K, C_out, chunks):
    mesh = plsc.VectorSubcoreMesh(core_axis_name="c", subcore_axis_name="s")

    @functools.partial(
        pl.kernel,
        out_type=jax.ShapeDtypeStruct((NP, C_out), jnp.float32),
        mesh=mesh,
        scratch_types=[
            pltpu.VMEM((K - 1, CB), jnp.int32),
            pltpu.VMEM((CB, C_out), jnp.float32),
            pltpu.SemaphoreType.DMA,
        ],
    )
    def sc_gather(y_hbm, idx_hbm, s_hbm, idx_v, acc_v, sem):
        wid = lax.axis_index("s") * NCORES + lax.axis_index("c")

        def chunk(t, carry):
            base = (wid * chunks + t) * CB
            pltpu.sync_copy(idx_hbm.at[:, pl.ds(base, CB)], idx_v)

            def off(kk, c2):
                sl = pl.ds(kk * 16, 16)
                for j in range(K - 1):
                    idx_v[j, sl] = idx_v[j, sl] + jnp.int32((j + 1) * NP)
                return c2

            lax.fori_loop(0, CB // 16, off, 0)
            pltpu.sync_copy(y_hbm.at[pl.ds(base, CB)], acc_v)
            copies = [
                pltpu.async_copy(y_hbm.at[idx_v.at[j]], acc_v, sem, add=True)
                for j in range(K - 1)
            ]
            for c in copies:
                c.wait()
            pltpu.sync_copy(acc_v, s_hbm.at[pl.ds(base, CB)])
            return carry

        lax.fori_loop(0, chunks, chunk, 0)

    return sc_gather


def kernel(x, gemm_vs, W, b):
    Bsz, C, N = x.shape
    K = gemm_vs.shape[-1] + 1
    C_out = W.shape[0]
    NP = ((N + NW * CB - 1) // (NW * CB)) * (NW * CB)
    chunks = NP // (NW * CB)
    nblk = pl.cdiv(N, CBN)

    Wr = jnp.transpose(W, (2, 1, 0)).astype(jnp.bfloat16)
    b2 = b.reshape(1, C_out)
    idxT = jnp.transpose(gemm_vs[0])
    idxT = jnp.pad(idxT, ((0, 0), (0, NP - N)))

    Y = pl.pallas_call(
        _mm_body,
        grid=(nblk, K),
        in_specs=[
            pl.BlockSpec((1, C, CBN), lambda nb, j: (0, 0, nb)),
            pl.BlockSpec((1, C, C_out), lambda nb, j: (j, 0, 0)),
            pl.BlockSpec((1, C_out), lambda nb, j: (0, 0)),
        ],
        out_specs=pl.BlockSpec((CBN, C_out), lambda nb, j: (j * (NP // CBN) + nb, 0)),
        out_shape=jax.ShapeDtypeStruct((K * NP, C_out), jnp.float32),
    )(x, Wr, b2)

    S = _make_sc_gather(NP, K, C_out, chunks)(Y, idxT)

    out = pl.pallas_call(
        _tr_body,
        grid=(nblk,),
        in_specs=[pl.BlockSpec((CBN, C_out), lambda nb: (nb, 0))],
        out_specs=pl.BlockSpec((1, C_out, CBN), lambda nb: (0, 0, nb)),
        out_shape=jax.ShapeDtypeStruct((1, C_out, N), jnp.float32),
    )(S)

    return out[..., None]


# speedup vs baseline: 1.4102x; 1.0194x over previous
import functools

import jax
import jax.numpy as jnp
from jax import lax
from jax.experimental import pallas as pl
from jax.experimental.pallas import tpu as pltpu
from jax.experimental.pallas import tpu_sc as plsc

NCORES = 2
NSUB = 16
NW = NCORES * NSUB
CB = 128
CBN = 512


def _mm_body(x_ref, w_ref, b_ref, y_ref):
    j = pl.program_id(1)
    acc = lax.dot_general(
        x_ref[0].astype(jnp.bfloat16), w_ref[0], (((0,), (0,)), ((), ())),
        preferred_element_type=jnp.float32)
    bias = jnp.where(j == 0, b_ref[...], jnp.zeros_like(b_ref[...]))
    y_ref[...] = acc + bias


def _tr_body(s_ref, o_ref):
    o_ref[0] = s_ref[...].T


def _make_sc_gather(NP, K, C_out, chunks):
    mesh = plsc.VectorSubcoreMesh(core_axis_name="c", subcore_axis_name="s")

    @functools.partial(
        pl.kernel,
        out_type=jax.ShapeDtypeStruct((NP, C_out), jnp.float32),
        mesh=mesh,
        scratch_types=[
            pltpu.VMEM((2, K, CB), jnp.int32),
            pltpu.VMEM((2, CB, C_out), jnp.float32),
            pltpu.SemaphoreType.DMA,
            pltpu.SemaphoreType.DMA,
            pltpu.SemaphoreType.DMA,
        ],
    )
    def sc_gather(y_hbm, idx_hbm, s_hbm, idx_v, acc_v, sem_i, sem_g, sem_o):
        wid = lax.axis_index("s") * NCORES + lax.axis_index("c")
        base0 = wid * chunks * CB

        def fire_idx(t, p):
            pltpu.async_copy(
                idx_hbm.at[:, pl.ds(base0 + t * CB, CB)],
                idx_v.at[p, pl.ds(1, K - 1)], sem_i)

        def body(t, p):
            base = base0 + t * CB
            pltpu.make_async_copy(
                idx_hbm.at[:, pl.ds(0, CB)],
                idx_v.at[p, pl.ds(1, K - 1)], sem_i).wait()
            @pl.when(t >= 2)
            def _():
                pltpu.make_async_copy(
                    acc_v.at[p], s_hbm.at[pl.ds(0, CB)], sem_o).wait()

            iota16 = lax.iota(jnp.int32, 16)

            def prep(kk, c2):
                sl = pl.ds(kk * 16, 16)
                idx_v[p, 0, sl] = base + kk * 16 + iota16
                for j in range(1, K):
                    idx_v[p, j, sl] = idx_v[p, j, sl] + jnp.int32(j * NP)
                return c2

            lax.fori_loop(0, CB // 16, prep, 0)

            zvec = jnp.zeros((16,), jnp.float32)

            def zero(i, c2):
                for v in range(C_out // 16):
                    acc_v[p, i, pl.ds(v * 16, 16)] = zvec
                return c2

            lax.fori_loop(0, CB, zero, 0)
            copies = [
                pltpu.async_copy(
                    y_hbm.at[idx_v.at[p, j]], acc_v.at[p], sem_g, add=True)
                for j in range(K)
            ]
            @pl.when(t + 1 < chunks)
            def _():
                fire_idx(t + 1, 1 - p)

            for c in copies:
                c.wait()
            pltpu.async_copy(acc_v.at[p], s_hbm.at[pl.ds(base, CB)], sem_o)

        fire_idx(0, 0)

        def pair(tt, carry):
            for p in range(2):
                t = tt * 2 + p

                @pl.when(t < chunks)
                def _():
                    body(t, p)

            return carry

        lax.fori_loop(0, (chunks + 1) // 2, pair, 0)
        for p in range(2):
            @pl.when(jnp.int32(chunks) > (1 - p))
            def _():
                pltpu.make_async_copy(
                    acc_v.at[p], s_hbm.at[pl.ds(0, CB)], sem_o).wait()

    return sc_gather


def kernel(x, gemm_vs, W, b):
    Bsz, C, N = x.shape
    K = gemm_vs.shape[-1] + 1
    C_out = W.shape[0]
    NP = ((N + NW * CB - 1) // (NW * CB)) * (NW * CB)
    chunks = NP // (NW * CB)
    nblk = pl.cdiv(N, CBN)

    Wr = jnp.transpose(W, (2, 1, 0)).astype(jnp.bfloat16)
    b2 = b.reshape(1, C_out)
    idxT = jnp.transpose(gemm_vs[0])
    idxT = jnp.pad(idxT, ((0, 0), (0, NP - N)))

    Y = pl.pallas_call(
        _mm_body,
        grid=(nblk, K),
        in_specs=[
            pl.BlockSpec((1, C, CBN), lambda nb, j: (0, 0, nb)),
            pl.BlockSpec((1, C, C_out), lambda nb, j: (j, 0, 0)),
            pl.BlockSpec((1, C_out), lambda nb, j: (0, 0)),
        ],
        out_specs=pl.BlockSpec((CBN, C_out), lambda nb, j: (j * (NP // CBN) + nb, 0)),
        out_shape=jax.ShapeDtypeStruct((K * NP, C_out), jnp.float32),
    )(x, Wr, b2)

    S = _make_sc_gather(NP, K, C_out, chunks)(Y, idxT)

    out = pl.pallas_call(
        _tr_body,
        grid=(nblk,),
        in_specs=[pl.BlockSpec((CBN, C_out), lambda nb: (nb, 0))],
        out_specs=pl.BlockSpec((1, C_out, CBN), lambda nb: (0, 0, nb)),
        out_shape=jax.ShapeDtypeStruct((1, C_out, N), jnp.float32),
    )(S)

    return out[..., None]


# speedup vs baseline: 2.4193x; 1.7155x over previous
import functools

import jax
import jax.numpy as jnp
from jax import lax
from jax.experimental import pallas as pl
from jax.experimental.pallas import tpu as pltpu
from jax.experimental.pallas import tpu_sc as plsc

NCORES = 2
NSUB = 16
NW = NCORES * NSUB
CB = 128
CBN = 512


def _mm_body(x_ref, w_ref, b_ref, y_ref):
    K = w_ref.shape[0]
    xb = x_ref[0].astype(jnp.bfloat16)
    for j in range(K):
        acc = lax.dot_general(
            xb, w_ref[j], (((0,), (0,)), ((), ())),
            preferred_element_type=jnp.float32)
        if j == 0:
            acc = acc + b_ref[...]
        y_ref[j] = acc


def _tr_body(s_ref, o_ref):
    o_ref[0] = s_ref[...].T


def _make_sc_gather(NP, K, C_out, chunks):
    mesh = plsc.VectorSubcoreMesh(core_axis_name="c", subcore_axis_name="s")
    NBR = K - 1

    @functools.partial(
        pl.kernel,
        out_type=jax.ShapeDtypeStruct((NP, C_out), jnp.float32),
        mesh=mesh,
        scratch_types=[
            pltpu.VMEM((2, K, CB), jnp.int32),
            pltpu.VMEM((2, CB, C_out), jnp.float32),
            pltpu.SemaphoreType.DMA,
            pltpu.SemaphoreType.DMA,
            pltpu.SemaphoreType.DMA,
        ],
    )
    def sc_gather(y_hbm, idx_hbm, s_hbm, idx_v, acc_v, sem_i, sem_g, sem_o):
        wid = lax.axis_index("s") * NCORES + lax.axis_index("c")
        base0 = wid * chunks * CB

        def fire_idx(t, p):
            pltpu.async_copy(
                idx_hbm.at[:, pl.ds(base0 + t * CB, CB)],
                idx_v.at[p, pl.ds(1, K - 1)], sem_i)

        def body(t, p):
            base = base0 + t * CB
            pltpu.make_async_copy(
                idx_hbm.at[:, pl.ds(0, CB)],
                idx_v.at[p, pl.ds(1, K - 1)], sem_i).wait()
            @pl.when(t >= 2)
            def _():
                pltpu.make_async_copy(
                    acc_v.at[p], s_hbm.at[pl.ds(0, CB)], sem_o).wait()

            iota16 = lax.iota(jnp.int32, 16)

            def prep(kk, c2):
                sl = pl.ds(kk * 16, 16)
                idx_v[p, 0, sl] = base + kk * 16 + iota16
                for j in range(1, K):
                    idx_v[p, j, sl] = idx_v[p, j, sl] + jnp.int32(j * NP)
                return c2

            lax.fori_loop(0, CB // 16, prep, 0)

            zvec = jnp.zeros((16,), jnp.float32)

            def zero(i, c2):
                for v in range(C_out // 16):
                    acc_v[p, i, pl.ds(v * 16, 16)] = zvec
                return c2

            lax.fori_loop(0, CB, zero, 0)
            copies = [
                pltpu.async_copy(
                    y_hbm.at[idx_v.at[p, j]], acc_v.at[p], sem_g, add=True)
                for j in range(K)
            ]
            @pl.when(t + 1 < chunks)
            def _():
                fire_idx(t + 1, 1 - p)

            for c in copies:
                c.wait()
            pltpu.async_copy(acc_v.at[p], s_hbm.at[pl.ds(base, CB)], sem_o)

        fire_idx(0, 0)

        def pair(tt, carry):
            for p in range(2):
                t = tt * 2 + p

                @pl.when(t < chunks)
                def _():
                    body(t, p)

            return carry

        lax.fori_loop(0, (chunks + 1) // 2, pair, 0)
        for p in range(2):
            @pl.when(jnp.int32(chunks) > (1 - p))
            def _():
                pltpu.make_async_copy(
                    acc_v.at[p], s_hbm.at[pl.ds(0, CB)], sem_o).wait()

    return sc_gather


def kernel(x, gemm_vs, W, b):
    Bsz, C, N = x.shape
    K = gemm_vs.shape[-1] + 1
    C_out = W.shape[0]
    NP = ((N + NW * CB - 1) // (NW * CB)) * (NW * CB)
    chunks = NP // (NW * CB)
    nblk = pl.cdiv(N, CBN)

    Wr = jnp.transpose(W, (2, 1, 0)).astype(jnp.bfloat16)
    b2 = b.reshape(1, C_out)
    idxT = jnp.transpose(gemm_vs[0])
    idxT = jnp.pad(idxT, ((0, 0), (0, NP - N)))

    Y = pl.pallas_call(
        _mm_body,
        grid=(nblk,),
        in_specs=[
            pl.BlockSpec((1, C, CBN), lambda nb: (0, 0, nb)),
            pl.BlockSpec((K, C, C_out), lambda nb: (0, 0, 0)),
            pl.BlockSpec((1, C_out), lambda nb: (0, 0)),
        ],
        out_specs=pl.BlockSpec((K, CBN, C_out), lambda nb: (0, nb, 0)),
        out_shape=jax.ShapeDtypeStruct((K, NP, C_out), jnp.float32),
    )(x, Wr, b2)

    S = _make_sc_gather(NP, K, C_out, chunks)(
        Y.reshape(K * NP, C_out), idxT)

    out = pl.pallas_call(
        _tr_body,
        grid=(nblk,),
        in_specs=[pl.BlockSpec((CBN, C_out), lambda nb: (nb, 0))],
        out_specs=pl.BlockSpec((1, C_out, CBN), lambda nb: (0, 0, nb)),
        out_shape=jax.ShapeDtypeStruct((1, C_out, N), jnp.float32),
    )(S)

    return out[..., None]


# speedup vs baseline: 3.4274x; 1.4167x over previous
import functools

import jax
import jax.numpy as jnp
from jax import lax
from jax.experimental import pallas as pl
from jax.experimental.pallas import tpu as pltpu
from jax.experimental.pallas import tpu_sc as plsc

NCORES = 2
NSUB = 16
NW = NCORES * NSUB
CB = 128
CBN = 512


def _mm_body(N, NP, x_ref, w_ref, b_ref, gv_ref, y_ref, t_ref):
    K = w_ref.shape[0]
    xb = x_ref[0].astype(jnp.bfloat16)
    for j in range(K):
        acc = lax.dot_general(
            xb, w_ref[j], (((0,), (0,)), ((), ())),
            preferred_element_type=jnp.float32)
        if j == 0:
            acc = acc + b_ref[...]
        y_ref[j] = acc
    offs = (lax.broadcasted_iota(jnp.int32, (K - 1, t_ref.shape[1]), 0) + 1) * NP
    t_ref[...] = jnp.clip(gv_ref[0], 0, N - 1).T + offs


def _tr_body(s_ref, y0_ref, o_ref):
    o_ref[0] = (s_ref[...] + y0_ref[0]).T


def _make_sc_gather(NP, K, C_out, chunks):
    mesh = plsc.VectorSubcoreMesh(core_axis_name="c", subcore_axis_name="s")
    NBR = K - 1

    @functools.partial(
        pl.kernel,
        out_type=jax.ShapeDtypeStruct((NP, C_out), jnp.float32),
        mesh=mesh,
        scratch_types=[
            pltpu.VMEM((2, K - 1, CB), jnp.int32),
            pltpu.VMEM((2, CB, C_out), jnp.float32),
            pltpu.SemaphoreType.DMA,
            pltpu.SemaphoreType.DMA,
            pltpu.SemaphoreType.DMA,
        ],
    )
    def sc_gather(y_hbm, idx_hbm, s_hbm, idx_v, acc_v, sem_i, sem_g, sem_o):
        wid = lax.axis_index("s") * NCORES + lax.axis_index("c")
        base0 = wid * chunks * CB

        def fire_idx(t, p):
            pltpu.async_copy(
                idx_hbm.at[:, pl.ds(base0 + t * CB, CB)], idx_v.at[p], sem_i)

        def body(t, p):
            base = base0 + t * CB
            pltpu.make_async_copy(
                idx_hbm.at[:, pl.ds(0, CB)], idx_v.at[p], sem_i).wait()
            @pl.when(t >= 2)
            def _():
                pltpu.make_async_copy(
                    acc_v.at[p], s_hbm.at[pl.ds(0, CB)], sem_o).wait()

            zvec = jnp.zeros((16,), jnp.float32)

            def zero(i, c2):
                for v in range(C_out // 16):
                    acc_v[p, i, pl.ds(v * 16, 16)] = zvec
                return c2

            lax.fori_loop(0, CB, zero, 0)
            copies = [
                pltpu.async_copy(
                    y_hbm.at[idx_v.at[p, j]], acc_v.at[p], sem_g, add=True)
                for j in range(K - 1)
            ]
            @pl.when(t + 1 < chunks)
            def _():
                fire_idx(t + 1, 1 - p)

            for c in copies:
                c.wait()
            pltpu.async_copy(acc_v.at[p], s_hbm.at[pl.ds(base, CB)], sem_o)

        fire_idx(0, 0)

        def pair(tt, carry):
            for p in range(2):
                t = tt * 2 + p

                @pl.when(t < chunks)
                def _():
                    body(t, p)

            return carry

        lax.fori_loop(0, (chunks + 1) // 2, pair, 0)
        for p in range(2):
            @pl.when(jnp.int32(chunks) > (1 - p))
            def _():
                pltpu.make_async_copy(
                    acc_v.at[p], s_hbm.at[pl.ds(0, CB)], sem_o).wait()

    return sc_gather


def kernel(x, gemm_vs, W, b):
    Bsz, C, N = x.shape
    K = gemm_vs.shape[-1] + 1
    C_out = W.shape[0]
    NP = ((N + NW * CB - 1) // (NW * CB)) * (NW * CB)
    chunks = NP // (NW * CB)
    nblk = pl.cdiv(N, CBN)

    Wr = jnp.transpose(W, (2, 1, 0)).astype(jnp.bfloat16)
    b2 = b.reshape(1, C_out)

    nblk_full = NP // CBN
    last_real = nblk - 1

    Y, idxT = pl.pallas_call(
        functools.partial(_mm_body, N, NP),
        grid=(nblk_full,),
        in_specs=[
            pl.BlockSpec((1, C, CBN),
                         lambda nb: (0, 0, jnp.minimum(nb, last_real))),
            pl.BlockSpec((K, C, C_out), lambda nb: (0, 0, 0)),
            pl.BlockSpec((1, C_out), lambda nb: (0, 0)),
            pl.BlockSpec((1, CBN, K - 1),
                         lambda nb: (0, jnp.minimum(nb, last_real), 0)),
        ],
        out_specs=[
            pl.BlockSpec((K, CBN, C_out), lambda nb: (0, nb, 0)),
            pl.BlockSpec((K - 1, CBN), lambda nb: (0, nb)),
        ],
        out_shape=[
            jax.ShapeDtypeStruct((K, NP, C_out), jnp.float32),
            jax.ShapeDtypeStruct((K - 1, NP), jnp.int32),
        ],
    )(x, Wr, b2, gemm_vs)

    S = _make_sc_gather(NP, K, C_out, chunks)(
        Y.reshape(K * NP, C_out), idxT)

    out = pl.pallas_call(
        _tr_body,
        grid=(nblk,),
        in_specs=[
            pl.BlockSpec((CBN, C_out), lambda nb: (nb, 0)),
            pl.BlockSpec((1, CBN, C_out), lambda nb: (0, nb, 0)),
        ],
        out_specs=pl.BlockSpec((1, C_out, CBN), lambda nb: (0, 0, nb)),
        out_shape=jax.ShapeDtypeStruct((1, C_out, N), jnp.float32),
    )(S, Y)

    return out[..., None]


# speedup vs baseline: 3.7781x; 1.1023x over previous
import functools

import jax
import jax.numpy as jnp
from jax import lax
from jax.experimental import pallas as pl
from jax.experimental.pallas import tpu as pltpu
from jax.experimental.pallas import tpu_sc as plsc

NCORES = 2
NSUB = 16
NW = NCORES * NSUB
CB = 128
CBN = 512


def _mm_body(N, NP, x_ref, w_ref, b_ref, gv_ref, y_ref, t_ref):
    K = w_ref.shape[0]
    xb = x_ref[0].astype(jnp.bfloat16)
    for j in range(K):
        acc = lax.dot_general(
            xb, w_ref[j], (((0,), (0,)), ((), ())),
            preferred_element_type=jnp.float32)
        if j == 0:
            acc = acc + b_ref[...]
        y_ref[j] = acc
    nb_cols = t_ref.shape[1] * t_ref.shape[2]
    offs = (lax.broadcasted_iota(jnp.int32, (K - 1, nb_cols), 0) + 1) * NP
    t = jnp.clip(gv_ref[0], 0, N - 1).T + offs
    t_ref[...] = t.reshape(t_ref.shape)


def _tr_body(s_ref, y0_ref, o_ref):
    o_ref[0] = (s_ref[...] + y0_ref[0]).T


def _make_sc_gather(NP, K, C_out, chunks):
    mesh = plsc.VectorSubcoreMesh(core_axis_name="c", subcore_axis_name="s")
    NBR = K - 1

    @functools.partial(
        pl.kernel,
        out_type=jax.ShapeDtypeStruct((NP, C_out), jnp.float32),
        mesh=mesh,
        scratch_types=[
            pltpu.VMEM((2, K - 1, CB), jnp.int32),
            pltpu.VMEM((2, CB, C_out), jnp.float32),
            pltpu.SemaphoreType.DMA,
            pltpu.SemaphoreType.DMA,
            pltpu.SemaphoreType.DMA,
        ],
    )
    def sc_gather(y_hbm, idx_hbm, s_hbm, idx_v, acc_v, sem_i, sem_g, sem_o):
        wid = lax.axis_index("s") * NCORES + lax.axis_index("c")
        base0 = wid * chunks * CB

        def fire_idx(t, p):
            r = (base0 + t * CB) // CB
            for j in range(K - 1):
                pltpu.async_copy(idx_hbm.at[j, r], idx_v.at[p, j], sem_i)

        def body(t, p):
            base = base0 + t * CB
            for j in range(K - 1):
                pltpu.make_async_copy(
                    idx_hbm.at[j, 0], idx_v.at[p, j], sem_i).wait()
            @pl.when(t >= 2)
            def _():
                pltpu.make_async_copy(
                    acc_v.at[p], s_hbm.at[pl.ds(0, CB)], sem_o).wait()

            zvec = jnp.zeros((16,), jnp.float32)

            def zero(i, c2):
                for v in range(C_out // 16):
                    acc_v[p, i, pl.ds(v * 16, 16)] = zvec
                return c2

            lax.fori_loop(0, CB, zero, 0)
            copies = [
                pltpu.async_copy(
                    y_hbm.at[idx_v.at[p, j]], acc_v.at[p], sem_g, add=True)
                for j in range(K - 1)
            ]
            @pl.when(t + 1 < chunks)
            def _():
                fire_idx(t + 1, 1 - p)

            for c in copies:
                c.wait()
            pltpu.async_copy(acc_v.at[p], s_hbm.at[pl.ds(base, CB)], sem_o)

        fire_idx(0, 0)

        def pair(tt, carry):
            for p in range(2):
                t = tt * 2 + p

                @pl.when(t < chunks)
                def _():
                    body(t, p)

            return carry

        lax.fori_loop(0, (chunks + 1) // 2, pair, 0)
        for p in range(2):
            @pl.when(jnp.int32(chunks) > (1 - p))
            def _():
                pltpu.make_async_copy(
                    acc_v.at[p], s_hbm.at[pl.ds(0, CB)], sem_o).wait()

    return sc_gather


def kernel(x, gemm_vs, W, b):
    Bsz, C, N = x.shape
    K = gemm_vs.shape[-1] + 1
    C_out = W.shape[0]
    NP = ((N + NW * CB - 1) // (NW * CB)) * (NW * CB)
    chunks = NP // (NW * CB)
    nblk = pl.cdiv(N, CBN)

    Wr = jnp.transpose(W, (2, 1, 0)).astype(jnp.bfloat16)
    b2 = b.reshape(1, C_out)

    CBN1 = 2 * CBN
    nblk_full = NP // CBN1
    last_real = pl.cdiv(N, CBN1) - 1

    Y, idxT = pl.pallas_call(
        functools.partial(_mm_body, N, NP),
        grid=(nblk_full,),
        in_specs=[
            pl.BlockSpec((1, C, CBN1),
                         lambda nb: (0, 0, jnp.minimum(nb, last_real))),
            pl.BlockSpec((K, C, C_out), lambda nb: (0, 0, 0)),
            pl.BlockSpec((1, C_out), lambda nb: (0, 0)),
            pl.BlockSpec((1, CBN1, K - 1),
                         lambda nb: (0, jnp.minimum(nb, last_real), 0)),
        ],
        out_specs=[
            pl.BlockSpec((K, CBN1, C_out), lambda nb: (0, nb, 0)),
            pl.BlockSpec((K - 1, CBN1 // CB, CB), lambda nb: (0, nb, 0)),
        ],
        out_shape=[
            jax.ShapeDtypeStruct((K, NP, C_out), jnp.float32),
            jax.ShapeDtypeStruct((K - 1, NP // CB, CB), jnp.int32),
        ],
    )(x, Wr, b2, gemm_vs)

    S = _make_sc_gather(NP, K, C_out, chunks)(
        Y.reshape(K * NP, C_out), idxT)

    out = pl.pallas_call(
        _tr_body,
        grid=(nblk,),
        in_specs=[
            pl.BlockSpec((CBN, C_out), lambda nb: (nb, 0)),
            pl.BlockSpec((1, CBN, C_out), lambda nb: (0, nb, 0)),
        ],
        out_specs=pl.BlockSpec((1, C_out, CBN), lambda nb: (0, 0, nb)),
        out_shape=jax.ShapeDtypeStruct((1, C_out, N), jnp.float32),
    )(S, Y)

    return out[..., None]


# speedup vs baseline: 3.8386x; 1.0160x over previous
import functools

import jax
import jax.numpy as jnp
from jax import lax
from jax.experimental import pallas as pl
from jax.experimental.pallas import tpu as pltpu
from jax.experimental.pallas import tpu_sc as plsc

NCORES = 2
NSUB = 16
NW = NCORES * NSUB
CB = 128
CBN = 512


def _mm_body(N, NP, x_ref, w_ref, b_ref, gv_ref, y_ref, t_ref):
    K = w_ref.shape[0]
    xb = x_ref[0].astype(jnp.bfloat16)
    for j in range(K):
        acc = lax.dot_general(
            xb, w_ref[j], (((0,), (0,)), ((), ())),
            preferred_element_type=jnp.float32)
        if j == 0:
            acc = acc + b_ref[...]
        y_ref[j] = acc
    nb_cols = t_ref.shape[1] * t_ref.shape[2]
    offs = (lax.broadcasted_iota(jnp.int32, (K - 1, nb_cols), 0) + 1) * NP
    t = jnp.clip(gv_ref[0], 0, N - 1).T + offs
    t_ref[...] = t.reshape(t_ref.shape)


def _tr_body(s_ref, y0_ref, o_ref):
    o_ref[0] = (s_ref[...] + y0_ref[0]).T


def _make_sc_gather(NP, K, C_out, chunks):
    mesh = plsc.VectorSubcoreMesh(core_axis_name="c", subcore_axis_name="s")
    NBR = K - 1

    @functools.partial(
        pl.kernel,
        out_type=jax.ShapeDtypeStruct((NP, C_out), jnp.float32),
        mesh=mesh,
        scratch_types=[
            pltpu.VMEM((2, K - 1, CB), jnp.int32),
            pltpu.VMEM((2, CB, C_out), jnp.float32),
            pltpu.SemaphoreType.DMA,
            pltpu.SemaphoreType.DMA,
            pltpu.SemaphoreType.DMA,
            pltpu.SemaphoreType.DMA,
        ],
    )
    def sc_gather(y_hbm, idx_hbm, s_hbm, idx_v, acc_v, sem_i, sem_g0, sem_g1,
                  sem_o):
        sem_gs = (sem_g0, sem_g1)
        wid = lax.axis_index("s") * NCORES + lax.axis_index("c")
        base0 = wid * chunks * CB

        def fire_idx(t, p):
            r = (base0 + t * CB) // CB
            for j in range(K - 1):
                pltpu.async_copy(idx_hbm.at[j, r], idx_v.at[p, j], sem_i)

        def drain_gathers(p):
            for _j in range(K - 1):
                pltpu.make_async_copy(
                    y_hbm.at[pl.ds(0, CB)], acc_v.at[p], sem_gs[p]).wait()

        def body(t, p):
            base = base0 + t * CB
            for j in range(K - 1):
                pltpu.make_async_copy(
                    idx_hbm.at[j, 0], idx_v.at[p, j], sem_i).wait()
            @pl.when(t >= 2)
            def _():
                pltpu.make_async_copy(
                    acc_v.at[p], s_hbm.at[pl.ds(0, CB)], sem_o).wait()

            zvec = jnp.zeros((16,), jnp.float32)

            def zero(i, c2):
                for v in range(C_out // 16):
                    acc_v[p, i, pl.ds(v * 16, 16)] = zvec
                return c2

            lax.fori_loop(0, CB, zero, 0)
            for j in range(K - 1):
                pltpu.async_copy(
                    y_hbm.at[idx_v.at[p, j]], acc_v.at[p], sem_gs[p], add=True)
            @pl.when(t >= 1)
            def _():
                drain_gathers(1 - p)
                pltpu.async_copy(
                    acc_v.at[1 - p], s_hbm.at[pl.ds(base - CB, CB)], sem_o)

            @pl.when(t + 1 < chunks)
            def _():
                fire_idx(t + 1, 1 - p)

        fire_idx(0, 0)

        def pair(tt, carry):
            for p in range(2):
                t = tt * 2 + p

                @pl.when(t < chunks)
                def _():
                    body(t, p)

            return carry

        lax.fori_loop(0, (chunks + 1) // 2, pair, 0)
        p_last = (chunks - 1) % 2
        drain_gathers(p_last)
        pltpu.async_copy(
            acc_v.at[p_last],
            s_hbm.at[pl.ds((base0 + (chunks - 1) * CB), CB)], sem_o)
        for _p in range(2):
            pltpu.make_async_copy(
                acc_v.at[_p], s_hbm.at[pl.ds(0, CB)], sem_o).wait()

    return sc_gather


def kernel(x, gemm_vs, W, b):
    Bsz, C, N = x.shape
    K = gemm_vs.shape[-1] + 1
    C_out = W.shape[0]
    NP = ((N + NW * CB - 1) // (NW * CB)) * (NW * CB)
    chunks = NP // (NW * CB)
    nblk = pl.cdiv(N, CBN)

    Wr = jnp.transpose(W, (2, 1, 0)).astype(jnp.bfloat16)
    b2 = b.reshape(1, C_out)

    CBN1 = 2 * CBN
    nblk_full = NP // CBN1
    last_real = pl.cdiv(N, CBN1) - 1

    Y, idxT = pl.pallas_call(
        functools.partial(_mm_body, N, NP),
        grid=(nblk_full,),
        in_specs=[
            pl.BlockSpec((1, C, CBN1),
                         lambda nb: (0, 0, jnp.minimum(nb, last_real))),
            pl.BlockSpec((K, C, C_out), lambda nb: (0, 0, 0)),
            pl.BlockSpec((1, C_out), lambda nb: (0, 0)),
            pl.BlockSpec((1, CBN1, K - 1),
                         lambda nb: (0, jnp.minimum(nb, last_real), 0)),
        ],
        out_specs=[
            pl.BlockSpec((K, CBN1, C_out), lambda nb: (0, nb, 0)),
            pl.BlockSpec((K - 1, CBN1 // CB, CB), lambda nb: (0, nb, 0)),
        ],
        out_shape=[
            jax.ShapeDtypeStruct((K, NP, C_out), jnp.float32),
            jax.ShapeDtypeStruct((K - 1, NP // CB, CB), jnp.int32),
        ],
    )(x, Wr, b2, gemm_vs)

    S = _make_sc_gather(NP, K, C_out, chunks)(
        Y.reshape(K * NP, C_out), idxT)

    out = pl.pallas_call(
        _tr_body,
        grid=(nblk,),
        in_specs=[
            pl.BlockSpec((CBN, C_out), lambda nb: (nb, 0)),
            pl.BlockSpec((1, CBN, C_out), lambda nb: (0, nb, 0)),
        ],
        out_specs=pl.BlockSpec((1, C_out, CBN), lambda nb: (0, 0, nb)),
        out_shape=jax.ShapeDtypeStruct((1, C_out, N), jnp.float32),
    )(S, Y)

    return out[..., None]
